# baseline (speedup 1.0000x reference)
import functools

import jax
import jax.numpy as jnp
from jax import lax
from jax.experimental import pallas as pl
from jax.experimental.pallas import tpu as pltpu
from jax.experimental.pallas import tpu_sc as plsc

NC, NS, L = 2, 16, 16
NW = NC * NS
B = 16384
D = 128
BPW = B // NW
C = 128
G = BPW // C
NB = 2

R = 1024
SECONDS_PER_DAY = 86400.0


def _gather_body(ids_hbm, table_hbm, out_hbm,
                 idx_v, rows_v, isem, gsem0, gsem1, osem0, osem1):
    wid = lax.axis_index("s") * NC + lax.axis_index("c")
    base = wid * BPW
    gsems = (gsem0, gsem1)
    osems = (osem0, osem1)
    ih = [None] * G
    gh = [None] * G
    oh = [None] * G

    def start_idx(g):
        ih[g] = pltpu.async_copy(
            ids_hbm.at[pl.ds(base + g * C, C)], idx_v.at[g % NB], isem)

    def start_gather(g):
        nb = g % NB
        gh[g] = pltpu.async_copy(
            table_hbm.at[idx_v.at[nb]], rows_v.at[nb], gsems[nb])

    start_idx(0)
    if G > 1:
        start_idx(1)
    ih[0].wait()
    start_gather(0)
    for g in range(G):
        nb = g % NB
        if g + 1 < G:
            ih[g + 1].wait()
            start_gather(g + 1)
        if g + NB < G:
            start_idx(g + NB)
        gh[g].wait()
        if g >= NB:
            oh[g - NB].wait()
        oh[g] = pltpu.async_copy(
            rows_v.at[nb], out_hbm.at[pl.ds(base + g * C, C)], osems[nb])
    for g in range(max(0, G - NB), G):
        oh[g].wait()


_gather_call = pl.kernel(
    _gather_body,
    out_type=jax.ShapeDtypeStruct((B, D), jnp.float32),
    mesh=plsc.VectorSubcoreMesh(core_axis_name="c", subcore_axis_name="s"),
    scratch_types=[
        pltpu.VMEM((NB, C), jnp.int32),
        pltpu.VMEM((NB, C, D), jnp.float32),
        pltpu.SemaphoreType.DMA,
        pltpu.SemaphoreType.DMA,
        pltpu.SemaphoreType.DMA,
        pltpu.SemaphoreType.DMA,
        pltpu.SemaphoreType.DMA,
    ],
)


def _math_body(ab_ref, lam_ref, dt_ref, rc_ref, prof_ref, out_ref):
    a = ab_ref[0, 0]
    g = ab_ref[0, 1]
    lam = jnp.clip(lam_ref[...], 0.005, 0.05)
    denom = (1.0 + a * jnp.log1p(rc_ref[...])) \
        * (1.0 + g * jnp.clip(prof_ref[...], 0.0, 1.0))[:, None]
    z = lam * dt_ref[...] * (-1.0 / SECONDS_PER_DAY)
    out_ref[...] = jnp.exp(z / denom)


_math_call = pl.pallas_call(
    _math_body,
    out_shape=jax.ShapeDtypeStruct((B, D), jnp.float32),
    grid=(B // R,),
    in_specs=[
        pl.BlockSpec(memory_space=pltpu.SMEM),
        pl.BlockSpec((R, D), lambda i: (i, 0)),
        pl.BlockSpec((R, D), lambda i: (i, 0)),
        pl.BlockSpec((R, D), lambda i: (i, 0)),
        pl.BlockSpec((R,), lambda i: (i,)),
    ],
    out_specs=pl.BlockSpec((R, D), lambda i: (i, 0)),
)


def kernel(student_ids, delta_t, review_count, proficiency, lambda_table,
           alpha_logit, gamma_logit):
    alpha = jax.nn.sigmoid(alpha_logit) * 1.9 + 0.1
    gamma = jax.nn.sigmoid(gamma_logit) * 2.9 + 0.1
    ab = jnp.stack([alpha, gamma]).reshape(1, 2)
    ids = student_ids.astype(jnp.int32)
    lam = _gather_call(ids, lambda_table)
    return _math_call(ab, lam, delta_t, review_count, proficiency)


# speedup vs baseline: 1.1463x; 1.1463x over previous
import functools

import jax
import jax.numpy as jnp
from jax import lax
from jax.experimental import pallas as pl
from jax.experimental.pallas import tpu as pltpu
from jax.experimental.pallas import tpu_sc as plsc

NC, NS, L = 2, 16, 16
NW = NC * NS
B = 16384
D = 128
BPW = B // NW
C = 128
G = BPW // C
NB = 2

R = 4096
SECONDS_PER_DAY = 86400.0


def _gather_body(ids_hbm, table_hbm, out_hbm,
                 idx_v, rows_v, isem, gsem0, gsem1, osem0, osem1):
    wid = lax.axis_index("s") * NC + lax.axis_index("c")
    base = wid * BPW
    gsems = (gsem0, gsem1)
    osems = (osem0, osem1)
    ih = [None] * G
    gh = [None] * G
    oh = [None] * G

    def start_idx(g):
        ih[g] = pltpu.async_copy(
            ids_hbm.at[pl.ds(base + g * C, C)], idx_v.at[g % NB], isem)

    def start_gather(g):
        nb = g % NB
        gh[g] = pltpu.async_copy(
            table_hbm.at[idx_v.at[nb]], rows_v.at[nb], gsems[nb])

    start_idx(0)
    if G > 1:
        start_idx(1)
    ih[0].wait()
    start_gather(0)
    for g in range(G):
        nb = g % NB
        if g + 1 < G:
            ih[g + 1].wait()
            start_gather(g + 1)
        if g + NB < G:
            start_idx(g + NB)
        gh[g].wait()
        if g >= NB:
            oh[g - NB].wait()
        oh[g] = pltpu.async_copy(
            rows_v.at[nb], out_hbm.at[pl.ds(base + g * C, C)], osems[nb])
    for g in range(max(0, G - NB), G):
        oh[g].wait()


_gather_call = pl.kernel(
    _gather_body,
    out_type=jax.ShapeDtypeStruct((B, D), jnp.float32),
    mesh=plsc.VectorSubcoreMesh(core_axis_name="c", subcore_axis_name="s"),
    scratch_types=[
        pltpu.VMEM((NB, C), jnp.int32),
        pltpu.VMEM((NB, C, D), jnp.float32),
        pltpu.SemaphoreType.DMA,
        pltpu.SemaphoreType.DMA,
        pltpu.SemaphoreType.DMA,
        pltpu.SemaphoreType.DMA,
        pltpu.SemaphoreType.DMA,
    ],
)


def _math_body(ab_ref, lam_ref, dt_ref, rc_ref, prof_ref, out_ref):
    a = ab_ref[0, 0]
    g = ab_ref[0, 1]
    lam = jnp.clip(lam_ref[...], 0.005, 0.05)
    denom = (1.0 + a * jnp.log1p(rc_ref[...])) \
        * (1.0 + g * jnp.clip(prof_ref[...], 0.0, 1.0))[:, None]
    z = lam * dt_ref[...] * (-1.0 / SECONDS_PER_DAY)
    out_ref[...] = jnp.exp(z / denom)


_math_call = pl.pallas_call(
    _math_body,
    out_shape=jax.ShapeDtypeStruct((B, D), jnp.float32),
    grid=(B // R,),
    in_specs=[
        pl.BlockSpec(memory_space=pltpu.SMEM),
        pl.BlockSpec((R, D), lambda i: (i, 0)),
        pl.BlockSpec((R, D), lambda i: (i, 0)),
        pl.BlockSpec((R, D), lambda i: (i, 0)),
        pl.BlockSpec((R,), lambda i: (i,)),
    ],
    out_specs=pl.BlockSpec((R, D), lambda i: (i, 0)),
)


def kernel(student_ids, delta_t, review_count, proficiency, lambda_table,
           alpha_logit, gamma_logit):
    alpha = jax.nn.sigmoid(alpha_logit) * 1.9 + 0.1
    gamma = jax.nn.sigmoid(gamma_logit) * 2.9 + 0.1
    ab = jnp.stack([alpha, gamma]).reshape(1, 2)
    ids = student_ids.astype(jnp.int32)
    lam = _gather_call(ids, lambda_table)
    return _math_call(ab, lam, delta_t, review_count, proficiency)


# speedup vs baseline: 1.1653x; 1.0165x over previous
import functools

import jax
import jax.numpy as jnp
from jax import lax
from jax.experimental import pallas as pl
from jax.experimental.pallas import tpu as pltpu
from jax.experimental.pallas import tpu_sc as plsc

NC, NS, L = 2, 16, 16
NW = NC * NS
B = 16384
D = 128
BPW = B // NW
C = 128
G = BPW // C
NB = 2

R = 8192
SECONDS_PER_DAY = 86400.0


def _gather_body(ids_hbm, table_hbm, out_hbm,
                 idx_v, rows_v, isem, gsem0, gsem1, osem0, osem1):
    wid = lax.axis_index("s") * NC + lax.axis_index("c")
    base = wid * BPW
    gsems = (gsem0, gsem1)
    osems = (osem0, osem1)
    ih = [None] * G
    gh = [None] * G
    oh = [None] * G

    def start_idx(g):
        ih[g] = pltpu.async_copy(
            ids_hbm.at[pl.ds(base + g * C, C)], idx_v.at[g % NB], isem)

    def start_gather(g):
        nb = g % NB
        gh[g] = pltpu.async_copy(
            table_hbm.at[idx_v.at[nb]], rows_v.at[nb], gsems[nb])

    start_idx(0)
    if G > 1:
        start_idx(1)
    ih[0].wait()
    start_gather(0)
    for g in range(G):
        nb = g % NB
        if g + 1 < G:
            ih[g + 1].wait()
            start_gather(g + 1)
        if g + NB < G:
            start_idx(g + NB)
        gh[g].wait()
        if g >= NB:
            oh[g - NB].wait()
        oh[g] = pltpu.async_copy(
            rows_v.at[nb], out_hbm.at[pl.ds(base + g * C, C)], osems[nb])
    for g in range(max(0, G - NB), G):
        oh[g].wait()


_gather_call = pl.kernel(
    _gather_body,
    out_type=jax.ShapeDtypeStruct((B, D), jnp.float32),
    mesh=plsc.VectorSubcoreMesh(core_axis_name="c", subcore_axis_name="s"),
    scratch_types=[
        pltpu.VMEM((NB, C), jnp.int32),
        pltpu.VMEM((NB, C, D), jnp.float32),
        pltpu.SemaphoreType.DMA,
        pltpu.SemaphoreType.DMA,
        pltpu.SemaphoreType.DMA,
        pltpu.SemaphoreType.DMA,
        pltpu.SemaphoreType.DMA,
    ],
)


def _math_body(ab_ref, lam_ref, dt_ref, rc_ref, prof_ref, out_ref):
    a = ab_ref[0, 0]
    g = ab_ref[0, 1]
    lam = jnp.clip(lam_ref[...], 0.005, 0.05)
    denom = (1.0 + a * jnp.log1p(rc_ref[...])) \
        * (1.0 + g * jnp.clip(prof_ref[...], 0.0, 1.0))[:, None]
    z = lam * dt_ref[...] * (-1.0 / SECONDS_PER_DAY)
    out_ref[...] = jnp.exp(z / denom)


_math_call = pl.pallas_call(
    _math_body,
    out_shape=jax.ShapeDtypeStruct((B, D), jnp.float32),
    grid=(B // R,),
    in_specs=[
        pl.BlockSpec(memory_space=pltpu.SMEM),
        pl.BlockSpec((R, D), lambda i: (i, 0)),
        pl.BlockSpec((R, D), lambda i: (i, 0)),
        pl.BlockSpec((R, D), lambda i: (i, 0)),
        pl.BlockSpec((R,), lambda i: (i,)),
    ],
    out_specs=pl.BlockSpec((R, D), lambda i: (i, 0)),
)


def kernel(student_ids, delta_t, review_count, proficiency, lambda_table,
           alpha_logit, gamma_logit):
    alpha = jax.nn.sigmoid(alpha_logit) * 1.9 + 0.1
    gamma = jax.nn.sigmoid(gamma_logit) * 2.9 + 0.1
    ab = jnp.stack([alpha, gamma]).reshape(1, 2)
    ids = student_ids.astype(jnp.int32)
    lam = _gather_call(ids, lambda_table)
    return _math_call(ab, lam, delta_t, review_count, proficiency)
